# baseline (speedup 1.0000x reference)
import jax
import jax.numpy as jnp
import numpy as np
from jax.experimental import pallas as pl
from jax.experimental.pallas import tpu as pltpu
from jax.sharding import Mesh, PartitionSpec as P

NUM_VISIBLE = 1024
NUM_EXPERTS = 16
HIDDEN_DIM = 256
ENERGY_SCALE = 3.0

BT = 2048


def _fused_kernel(x_ref, wenc_ref, benc_ref, we_ref, be_ref, it_ref,
                  p_ref, e_ref, l_ref):
    H = HIDDEN_DIM
    we = we_ref[...]
    wh = we[:, :H]
    wi = we[:, H:]
    enc = jnp.tanh(
        jnp.dot(x_ref[...], wenc_ref[...], preferred_element_type=jnp.float32)
        + benc_ref[...])
    s = jnp.sum(enc * wh, axis=1, keepdims=True)
    beta = jax.nn.softplus(it_ref[0, 0])
    en = ENERGY_SCALE * jnp.tanh(s + (wi + be_ref[0, 0]))
    lg = (-beta) * en
    m = jnp.max(lg, axis=-1, keepdims=True)
    ex = jnp.exp(lg - m)
    p_ref[...] = ex / jnp.sum(ex, axis=-1, keepdims=True)
    e_ref[...] = en
    l_ref[...] = lg


def _router(x, W_enc, b_enc2, we_row, be2, it2):
    B = x.shape[0]
    H = HIDDEN_DIM
    E = NUM_EXPERTS
    bt = min(BT, B)
    grid = (B // bt,)
    out_shape = [jax.ShapeDtypeStruct((B, E), jnp.float32)] * 3
    return pl.pallas_call(
        _fused_kernel,
        grid=grid,
        in_specs=[
            pl.BlockSpec((bt, NUM_VISIBLE), lambda i: (i, 0)),
            pl.BlockSpec((NUM_VISIBLE, H), lambda i: (0, 0)),
            pl.BlockSpec((1, H), lambda i: (0, 0)),
            pl.BlockSpec((1, H + E), lambda i: (0, 0)),
            pl.BlockSpec((1, 1), lambda i: (0, 0)),
            pl.BlockSpec((1, 1), lambda i: (0, 0)),
        ],
        out_specs=[pl.BlockSpec((bt, E), lambda i: (i, 0))] * 3,
        out_shape=out_shape,
        compiler_params=pltpu.CompilerParams(
            dimension_semantics=("parallel",)),
    )(x, W_enc, b_enc2, we_row, be2, it2)


def kernel(x, W_enc, b_enc, W_e, b_e, inv_temp):
    B = x.shape[0]
    H = HIDDEN_DIM
    E = NUM_EXPERTS
    we_row = W_e.reshape(1, H + E)
    b_enc2 = b_enc.reshape(1, H)
    be2 = b_e.reshape(1, 1)
    it2 = inv_temp.reshape(1, 1)

    devs = jax.devices()
    nd = len(devs)
    while nd > 1 and (B % nd != 0 or (B // nd) % 8 != 0):
        nd -= 1
    if nd <= 1:
        return tuple(_router(x, W_enc, b_enc2, we_row, be2, it2))

    mesh = Mesh(np.array(devs[:nd]), ("d",))
    sharded = jax.shard_map(
        _router,
        mesh=mesh,
        in_specs=(P("d", None), P(None, None), P(None, None),
                  P(None, None), P(None, None), P(None, None)),
        out_specs=(P("d", None),) * 3,
        check_vma=False,
    )
    return tuple(sharded(x, W_enc, b_enc2, we_row, be2, it2))


# speedup vs baseline: 18.0831x; 18.0831x over previous
import jax
import jax.numpy as jnp
from jax.experimental import pallas as pl
from jax.experimental.pallas import tpu as pltpu

NUM_VISIBLE = 1024
NUM_EXPERTS = 16
HIDDEN_DIM = 256
ENERGY_SCALE = 3.0

BT = 2048


def _fused_kernel(x_ref, wenc_ref, benc_ref, we_ref, be_ref, it_ref,
                  p_ref, e_ref, l_ref):
    H = HIDDEN_DIM
    we = we_ref[...]
    wh = we[:, :H]
    wi = we[:, H:]
    enc = jnp.tanh(
        jnp.dot(x_ref[...], wenc_ref[...], preferred_element_type=jnp.float32)
        + benc_ref[...])
    s = jnp.sum(enc * wh, axis=1, keepdims=True)
    beta = jax.nn.softplus(it_ref[0, 0])
    en = ENERGY_SCALE * jnp.tanh(s + (wi + be_ref[0, 0]))
    lg = (-beta) * en
    m = jnp.max(lg, axis=-1, keepdims=True)
    ex = jnp.exp(lg - m)
    p_ref[...] = ex / jnp.sum(ex, axis=-1, keepdims=True)
    e_ref[...] = en
    l_ref[...] = lg


def _router(x, W_enc, b_enc2, we_row, be2, it2):
    B = x.shape[0]
    H = HIDDEN_DIM
    E = NUM_EXPERTS
    bt = min(BT, B)
    grid = (B // bt,)
    out_shape = [jax.ShapeDtypeStruct((B, E), jnp.float32)] * 3
    return pl.pallas_call(
        _fused_kernel,
        grid=grid,
        in_specs=[
            pl.BlockSpec((bt, NUM_VISIBLE), lambda i: (i, 0)),
            pl.BlockSpec((NUM_VISIBLE, H), lambda i: (0, 0)),
            pl.BlockSpec((1, H), lambda i: (0, 0)),
            pl.BlockSpec((1, H + E), lambda i: (0, 0)),
            pl.BlockSpec((1, 1), lambda i: (0, 0)),
            pl.BlockSpec((1, 1), lambda i: (0, 0)),
        ],
        out_specs=[pl.BlockSpec((bt, E), lambda i: (i, 0))] * 3,
        out_shape=out_shape,
        compiler_params=pltpu.CompilerParams(
            dimension_semantics=("parallel",)),
    )(x, W_enc, b_enc2, we_row, be2, it2)


def kernel(x, W_enc, b_enc, W_e, b_e, inv_temp):
    B = x.shape[0]
    H = HIDDEN_DIM
    E = NUM_EXPERTS
    we_row = W_e.reshape(1, H + E)
    b_enc2 = b_enc.reshape(1, H)
    be2 = b_e.reshape(1, 1)
    it2 = inv_temp.reshape(1, 1)

    return tuple(_router(x, W_enc, b_enc2, we_row, be2, it2))
